# baseline (speedup 1.0000x reference)
import jax
import jax.numpy as jnp
from jax import lax
from jax.experimental import pallas as pl
from jax.experimental.pallas import tpu as pltpu
from jax.experimental.pallas import tpu_sc as plsc

_NVOC = 1000000
_NUM_BINS = _NVOC + 1
_EMB_DIM = 32
_BATCH = 16384
_LANES = 16
_NW = 32
_BPW = _BATCH // _NW
_NGRP = _BPW // _LANES
_NBLK_FULL = _NVOC // 128
_DEPTH = 4


def _hash_lanes(h, salt0, salt1):
    h = h * jnp.uint32(salt0) + jnp.uint32(salt1)
    h = h ^ (h >> jnp.uint32(16))
    h = h * jnp.uint32(0x45D9F3B)
    h = h ^ (h >> jnp.uint32(16))
    return (h % jnp.uint32(_NUM_BINS)).astype(jnp.int32)


def _iota16():
    return lax.broadcasted_iota(jnp.int32, (_LANES,), 0)


def _splat(v):
    return jnp.full((_LANES,), v, jnp.int32)


def _body(x_hbm, t1T_hbm, t2T_hbm, tail1_hbm, tail2_hbm, out_hbm,
          xv, idx1_v, idx2_v, slab, acc,
          sem0, sem1, sem2, sem3, sem4, sem5, sem6, sem7,
          sem8, sem9, sem10, sem11, sem12, sem13, sem14, sem15):
    w = lax.axis_index("s") * 2 + lax.axis_index("c")
    base = w * _BPW
    sems = (sem0, sem1, sem2, sem3, sem4, sem5, sem6, sem7,
            sem8, sem9, sem10, sem11, sem12, sem13, sem14, sem15)
    t_hbm = (t1T_hbm, t2T_hbm)
    idx_v = (idx1_v, idx2_v)

    pltpu.sync_copy(tail1_hbm, slab.at[16])
    pltpu.sync_copy(tail2_hbm, slab.at[17])

    pltpu.sync_copy(x_hbm.at[pl.ds(base, _BPW)], xv)

    def hash_step(k, _):
        xb = xv[pl.ds(k * _LANES, _LANES)].astype(jnp.uint32)
        idx1_v[pl.ds(k * _LANES, _LANES)] = _hash_lanes(xb, 6971, 7321)
        idx2_v[pl.ds(k * _LANES, _LANES)] = _hash_lanes(xb, 7723, 7507)
        return 0
    lax.fori_loop(0, _NGRP, hash_step, 0)

    def idx_scalar(t, goff, lane):
        return idx_v[t][pl.ds(goff, _LANES)][lane]

    def fire(i1, i2, par):
        for t, i in ((0, i1), (1, i2)):
            blk = jnp.minimum(
                lax.shift_right_logical(i, 7), _NBLK_FULL - 1)
            col0 = pl.multiple_of(blk * 128, 128)
            pltpu.async_copy(
                t_hbm[t].at[:, pl.ds(col0, 128)],
                slab.at[2 * par + t], sems[2 * par + t])

    def wait_pair(par):
        for t in range(2):
            pltpu.make_async_copy(
                t_hbm[t].at[:, pl.ds(0, 128)], slab.at[2 * par + t],
                sems[2 * par + t]).wait()

    def extract(i1, i2, b, par):
        vs = []
        for t, i in ((0, i1), (1, i2)):
            in_tail = i >= _NBLK_FULL * 128
            col = jnp.where(in_tail, i - _NBLK_FULL * 128, i & 127)
            src_major = jnp.where(in_tail, 16 + t, 2 * par + t)
            for half in range(2):
                dv = _iota16() + half * _LANES
                vs.append(plsc.load_gather(
                    slab, [_splat(src_major), dv, _splat(col)]))
        for half in range(2):
            prod = vs[half] * vs[2 + half]
            plsc.store_scatter(
                acc, [_iota16() + half * _LANES, _splat(b)], prod)

    for l0 in range(_DEPTH):
        fire(idx_scalar(0, 0, l0), idx_scalar(1, 0, l0), l0)

    def group_step(g, _):
        goff = g * _LANES
        goff_n = jnp.minimum(goff + _LANES, _BPW - _LANES)
        for lane in range(_LANES):
            b = goff + lane
            par = lane % _DEPTH
            wait_pair(par)
            extract(idx_scalar(0, goff, lane), idx_scalar(1, goff, lane),
                    b, par)
            if lane + _DEPTH < _LANES:
                pf = (goff, lane + _DEPTH)
            else:
                pf = (goff_n, lane + _DEPTH - _LANES)

            @pl.when(b + _DEPTH < _BPW)
            def _():
                fire(idx_scalar(0, pf[0], pf[1]),
                     idx_scalar(1, pf[0], pf[1]), par)
        return 0

    lax.fori_loop(0, _NGRP, group_step, 0)

    pltpu.sync_copy(acc, out_hbm.at[:, pl.ds(base, _BPW)])


@jax.jit
def kernel(x, table1, table2):
    mesh = plsc.VectorSubcoreMesh(core_axis_name="c", subcore_axis_name="s")
    tail1 = jnp.pad(table1[_NBLK_FULL * 128:].T, ((0, 0), (0, 63)))
    tail2 = jnp.pad(table2[_NBLK_FULL * 128:].T, ((0, 0), (0, 63)))

    run = pl.kernel(
        _body,
        mesh=mesh,
        compiler_params=pltpu.CompilerParams(
            use_tc_tiling_on_sc=True, needs_layout_passes=False),
        out_type=jax.ShapeDtypeStruct((_EMB_DIM, _BATCH), jnp.float32),
        scratch_types=[
            pltpu.VMEM((_BPW,), jnp.int32),
            pltpu.VMEM((_BPW,), jnp.int32),
            pltpu.VMEM((_BPW,), jnp.int32),
            pltpu.VMEM((18, _EMB_DIM, 128), jnp.float32),
            pltpu.VMEM((_EMB_DIM, _BPW), jnp.float32),
        ] + [pltpu.SemaphoreType.DMA] * 16,
    )
    out_t = run(x.astype(jnp.int32), table1.T, table2.T, tail1, tail2)
    return out_t.T


# speedup vs baseline: 1.0373x; 1.0373x over previous
import jax
import jax.numpy as jnp
from jax import lax
from jax.experimental import pallas as pl
from jax.experimental.pallas import tpu as pltpu
from jax.experimental.pallas import tpu_sc as plsc

_NVOC = 1000000
_NUM_BINS = _NVOC + 1
_EMB_DIM = 32
_BATCH = 16384
_LANES = 16
_NW = 32
_BPW = _BATCH // _NW
_NGRP = _BPW // _LANES
_NBLK_FULL = _NVOC // 128
_DEPTH = 8


def _hash_lanes(h, salt0, salt1):
    h = h * jnp.uint32(salt0) + jnp.uint32(salt1)
    h = h ^ (h >> jnp.uint32(16))
    h = h * jnp.uint32(0x45D9F3B)
    h = h ^ (h >> jnp.uint32(16))
    return (h % jnp.uint32(_NUM_BINS)).astype(jnp.int32)


def _iota16():
    return lax.broadcasted_iota(jnp.int32, (_LANES,), 0)


def _splat(v):
    return jnp.full((_LANES,), v, jnp.int32)


def _body(x_hbm, t1T_hbm, t2T_hbm, tail1_hbm, tail2_hbm, out_hbm,
          xv, idx1_v, idx2_v, slab, acc,
          sem0, sem1, sem2, sem3, sem4, sem5, sem6, sem7,
          sem8, sem9, sem10, sem11, sem12, sem13, sem14, sem15):
    w = lax.axis_index("s") * 2 + lax.axis_index("c")
    base = w * _BPW
    sems = (sem0, sem1, sem2, sem3, sem4, sem5, sem6, sem7,
            sem8, sem9, sem10, sem11, sem12, sem13, sem14, sem15)
    t_hbm = (t1T_hbm, t2T_hbm)
    idx_v = (idx1_v, idx2_v)

    pltpu.sync_copy(tail1_hbm, slab.at[16])
    pltpu.sync_copy(tail2_hbm, slab.at[17])

    pltpu.sync_copy(x_hbm.at[pl.ds(base, _BPW)], xv)

    def hash_step(k, _):
        xb = xv[pl.ds(k * _LANES, _LANES)].astype(jnp.uint32)
        idx1_v[pl.ds(k * _LANES, _LANES)] = _hash_lanes(xb, 6971, 7321)
        idx2_v[pl.ds(k * _LANES, _LANES)] = _hash_lanes(xb, 7723, 7507)
        return 0
    lax.fori_loop(0, _NGRP, hash_step, 0)

    def idx_scalar(t, goff, lane):
        return idx_v[t][pl.ds(goff, _LANES)][lane]

    def fire(i1, i2, par):
        for t, i in ((0, i1), (1, i2)):
            blk = jnp.minimum(
                lax.shift_right_logical(i, 7), _NBLK_FULL - 1)
            col0 = pl.multiple_of(blk * 128, 128)
            pltpu.async_copy(
                t_hbm[t].at[:, pl.ds(col0, 128)],
                slab.at[2 * par + t], sems[2 * par + t])

    def wait_pair(par):
        for t in range(2):
            pltpu.make_async_copy(
                t_hbm[t].at[:, pl.ds(0, 128)], slab.at[2 * par + t],
                sems[2 * par + t]).wait()

    def extract(i1, i2, b, par):
        vs = []
        for t, i in ((0, i1), (1, i2)):
            in_tail = i >= _NBLK_FULL * 128
            col = jnp.where(in_tail, i - _NBLK_FULL * 128, i & 127)
            src_major = jnp.where(in_tail, 16 + t, 2 * par + t)
            for half in range(2):
                dv = _iota16() + half * _LANES
                vs.append(plsc.load_gather(
                    slab, [_splat(src_major), dv, _splat(col)]))
        for half in range(2):
            prod = vs[half] * vs[2 + half]
            plsc.store_scatter(
                acc, [_iota16() + half * _LANES, _splat(b)], prod)

    for l0 in range(_DEPTH):
        fire(idx_scalar(0, 0, l0), idx_scalar(1, 0, l0), l0)

    def group_step(g, _):
        goff = g * _LANES
        goff_n = jnp.minimum(goff + _LANES, _BPW - _LANES)
        for lane in range(_LANES):
            b = goff + lane
            par = lane % _DEPTH
            wait_pair(par)
            extract(idx_scalar(0, goff, lane), idx_scalar(1, goff, lane),
                    b, par)
            if lane + _DEPTH < _LANES:
                pf = (goff, lane + _DEPTH)
            else:
                pf = (goff_n, lane + _DEPTH - _LANES)

            @pl.when(b + _DEPTH < _BPW)
            def _():
                fire(idx_scalar(0, pf[0], pf[1]),
                     idx_scalar(1, pf[0], pf[1]), par)
        return 0

    lax.fori_loop(0, _NGRP, group_step, 0)

    pltpu.sync_copy(acc, out_hbm.at[:, pl.ds(base, _BPW)])


@jax.jit
def kernel(x, table1, table2):
    mesh = plsc.VectorSubcoreMesh(core_axis_name="c", subcore_axis_name="s")
    tail1 = jnp.pad(table1[_NBLK_FULL * 128:].T, ((0, 0), (0, 63)))
    tail2 = jnp.pad(table2[_NBLK_FULL * 128:].T, ((0, 0), (0, 63)))

    run = pl.kernel(
        _body,
        mesh=mesh,
        compiler_params=pltpu.CompilerParams(
            use_tc_tiling_on_sc=True, needs_layout_passes=False),
        out_type=jax.ShapeDtypeStruct((_EMB_DIM, _BATCH), jnp.float32),
        scratch_types=[
            pltpu.VMEM((_BPW,), jnp.int32),
            pltpu.VMEM((_BPW,), jnp.int32),
            pltpu.VMEM((_BPW,), jnp.int32),
            pltpu.VMEM((18, _EMB_DIM, 128), jnp.float32),
            pltpu.VMEM((_EMB_DIM, _BPW), jnp.float32),
        ] + [pltpu.SemaphoreType.DMA] * 16,
    )
    out_t = run(x.astype(jnp.int32), table1.T, table2.T, tail1, tail2)
    return out_t.T
